# baseline (speedup 1.0000x reference)
import functools

import jax
import jax.numpy as jnp
from jax import lax
from jax.experimental import pallas as pl
from jax.experimental.pallas import tpu as pltpu
from jax.experimental.pallas import tpu_sc as plsc

B, S, D = 1, 2048, 768
E, K, F = 8, 2, 768
ALPHA, LIMIT = 1.702, 7.0

T = B * S
P2 = 2 * T
BLK = 256
NROWS = P2 + E * BLK
NB = NROWS // BLK
CHUNK = 512

_STOP = 99

NSUB = 32
PAIRS_PER = P2 // NSUB
TOK_PER = T // NSUB
HALF = TOK_PER // 2


def _router_body(x_ref, wr_ref, br_ref, pos_ref, wp_ref, be_ref, xi_ref,
                 act_ref):
    x = x_ref[...]
    logits = jnp.dot(x, wr_ref[...], preferred_element_type=jnp.float32)
    logits = logits + br_ref[...]
    ids = lax.broadcasted_iota(jnp.int32, (T, E), 1)
    m1 = jnp.max(logits, axis=1, keepdims=True)
    i1 = jnp.min(jnp.where(logits == m1, ids, E), axis=1, keepdims=True)
    mask1 = ids == i1
    l2 = jnp.where(mask1, -jnp.inf, logits)
    m2 = jnp.max(l2, axis=1, keepdims=True)
    i2 = jnp.min(jnp.where(l2 == m2, ids, E), axis=1, keepdims=True)
    mask2 = ids == i2
    tt = jnp.exp(m2 - m1)
    w1 = 1.0 / (1.0 + tt)
    w2 = 1.0 - w1

    oh = jnp.concatenate([mask1, mask2], axis=0).astype(jnp.float32)
    wp_ref[...] = jnp.concatenate([w1, w2], axis=0)

    ci = lax.broadcasted_iota(jnp.int32, (CHUNK, CHUNK), 0)
    cj = lax.broadcasted_iota(jnp.int32, (CHUNK, CHUNK), 1)
    tri = (cj <= ci).astype(jnp.float32)
    run = jnp.zeros((1, E), jnp.float32)
    chunks = []
    for c in range(P2 // CHUNK):
        seg = oh[c * CHUNK:(c + 1) * CHUNK]
        cs = jnp.dot(tri, seg, preferred_element_type=jnp.float32) + run
        run = cs[CHUNK - 1:CHUNK, :]
        chunks.append(cs)
    cums = jnp.concatenate(chunks, axis=0)

    counts = run.astype(jnp.int32)
    padded = ((counts + (BLK - 1)) // BLK) * BLK
    ei = lax.broadcasted_iota(jnp.int32, (E, E), 0)
    ej = lax.broadcasted_iota(jnp.int32, (E, E), 1)
    mex = (ei < ej).astype(jnp.float32)
    offs = jnp.dot(padded.astype(jnp.float32), mex,
                   preferred_element_type=jnp.float32)
    pos_ref[...] = jnp.sum(oh * (offs + cums - 1.0), axis=1,
                           keepdims=True).astype(jnp.int32)

    total = jnp.sum(padded)
    n_active = total // BLK
    bi = lax.broadcasted_iota(jnp.int32, (NB, 1), 0)
    bstart = bi * BLK
    offs_b = jnp.broadcast_to(offs.astype(jnp.int32), (NB, E))
    be = jnp.sum((offs_b <= bstart).astype(jnp.int32), axis=1,
                 keepdims=True) - 1
    act = (bstart < total).astype(jnp.int32)
    be_last = jnp.max(jnp.where(act == 1, be, -1), axis=0, keepdims=True)
    be_ref[...] = jnp.where(act == 1, be, be_last)
    xi_ref[...] = jnp.minimum(bi, n_active - 1)
    act_ref[...] = act


def _expert_body(be_ref, xi_ref, act_ref, xs_ref, wgu_ref, bgu_ref, wd2_ref,
                 bd_ref, y_ref):
    b = pl.program_id(0)
    h = pl.program_id(1)

    @pl.when(act_ref[b] == 1)
    def _():
        x = xs_ref[...]
        gu = jnp.dot(x, wgu_ref[0], preferred_element_type=jnp.float32)
        gu = gu + bgu_ref[0]
        g = pltpu.roll(gu, 1, 1)
        g = jnp.minimum(g, LIMIT)
        u = jnp.clip(gu, -LIMIT, LIMIT)
        actv = (u + 1.0) * (g * jax.nn.sigmoid(g * ALPHA))
        yp = jnp.dot(actv, wd2_ref[0], preferred_element_type=jnp.float32)

        @pl.when(h == 0)
        def _():
            y_ref[...] = yp + bd_ref[0]

        @pl.when(h == 1)
        def _():
            y_ref[...] += yp


def _dispatch_body(x_hbm, pos_hbm, xs_hbm, src_v, pos_v, rows_v, sem):
    wid = lax.axis_index("s") * 2 + lax.axis_index("c")
    base = wid * PAIRS_PER
    for i in range(PAIRS_PER // 16):
        v = base + i * 16 + lax.broadcasted_iota(jnp.int32, (16,), 0)
        src_v[pl.ds(i * 16, 16)] = lax.rem(v, T)
    pltpu.sync_copy(pos_hbm.at[pl.ds(base, PAIRS_PER)], pos_v)
    pltpu.async_copy(x_hbm.at[src_v], rows_v, sem).wait()
    pltpu.async_copy(rows_v, xs_hbm.at[pos_v], sem).wait()


def _combine_body(y_hbm, pos_hbm, out0_hbm, out1_hbm, pos_v, rows_v, sem):
    wid = lax.axis_index("s") * 2 + lax.axis_index("c")
    t0 = wid * TOK_PER
    pltpu.sync_copy(pos_hbm.at[pl.ds(t0, TOK_PER)], pos_v)
    pltpu.async_copy(y_hbm.at[pos_v], rows_v, sem).wait()
    pltpu.sync_copy(rows_v, out0_hbm.at[pl.ds(t0, TOK_PER)])
    pltpu.sync_copy(pos_hbm.at[pl.ds(T + t0, TOK_PER)], pos_v)
    pltpu.async_copy(y_hbm.at[pos_v], rows_v, sem).wait()
    pltpu.sync_copy(rows_v, out1_hbm.at[pl.ds(t0, TOK_PER)])


def _mix_body(w0_ref, w1_ref, a_ref, b_ref, o_ref):
    o_ref[...] = w0_ref[...] * a_ref[...] + w1_ref[...] * b_ref[...]


@functools.lru_cache(maxsize=2)
def _build(interpret: bool = False):
    mesh = plsc.VectorSubcoreMesh(core_axis_name="c", subcore_axis_name="s")

    router = pl.pallas_call(
        _router_body,
        out_shape=[
            jax.ShapeDtypeStruct((P2, 1), jnp.int32),
            jax.ShapeDtypeStruct((P2, 1), jnp.float32),
            jax.ShapeDtypeStruct((NB, 1), jnp.int32),
            jax.ShapeDtypeStruct((NB, 1), jnp.int32),
            jax.ShapeDtypeStruct((NB, 1), jnp.int32),
        ],
        interpret=interpret,
    )

    dispatch = functools.partial(
        pl.kernel,
        out_type=jax.ShapeDtypeStruct((NROWS, D), jnp.float32),
        mesh=mesh,
        scratch_types=[
            pltpu.VMEM((PAIRS_PER,), jnp.int32),
            pltpu.VMEM((PAIRS_PER,), jnp.int32),
            pltpu.VMEM((PAIRS_PER, D), jnp.float32),
            pltpu.SemaphoreType.DMA,
        ],
        interpret=interpret,
    )(_dispatch_body)

    grid_spec = pltpu.PrefetchScalarGridSpec(
        num_scalar_prefetch=3,
        grid=(NB, 2),
        in_specs=[
            pl.BlockSpec((BLK, D), lambda b, h, be, xi, act: (xi[b], 0)),
            pl.BlockSpec((1, D, F), lambda b, h, be, xi, act: (be[b], 0, h)),
            pl.BlockSpec((1, 1, F), lambda b, h, be, xi, act: (be[b], 0, h)),
            pl.BlockSpec((1, F, D), lambda b, h, be, xi, act: (be[b], h, 0)),
            pl.BlockSpec((1, 1, D), lambda b, h, be, xi, act: (be[b], 0, 0)),
        ],
        out_specs=pl.BlockSpec((BLK, D), lambda b, h, be, xi, act: (b, 0)),
    )
    expert = pl.pallas_call(
        _expert_body,
        grid_spec=grid_spec,
        out_shape=jax.ShapeDtypeStruct((NROWS, D), jnp.float32),
        interpret=interpret,
    )

    combine = functools.partial(
        pl.kernel,
        out_type=[
            jax.ShapeDtypeStruct((T, D), jnp.float32),
            jax.ShapeDtypeStruct((T, D), jnp.float32),
        ],
        mesh=mesh,
        scratch_types=[
            pltpu.VMEM((TOK_PER,), jnp.int32),
            pltpu.VMEM((TOK_PER, D), jnp.float32),
            pltpu.SemaphoreType.DMA,
        ],
        interpret=interpret,
    )(_combine_body)

    mix = pl.pallas_call(
        _mix_body,
        out_shape=jax.ShapeDtypeStruct((T, D), jnp.float32),
        interpret=interpret,
    )

    def run(hidden_states, router_weight, router_bias, gate_up_proj,
            gate_up_proj_bias, down_proj, down_proj_bias):
        x = hidden_states.reshape(T, D)
        bgu = gate_up_proj_bias.reshape(E, 1, 2 * F)
        wd2 = jnp.concatenate(
            [jnp.zeros_like(down_proj)[:, :, None, :],
             down_proj[:, :, None, :]], axis=2).reshape(E, 2 * F, D)
        bd = down_proj_bias.reshape(E, 1, D)
        br = router_bias.reshape(1, E)

        pos2, wp2, be2, xi2, act2 = router(x, router_weight, br)
        if _STOP == 1:
            return (jnp.sum(wd2) + jnp.sum(pos2.astype(jnp.float32)) +
                    jnp.zeros((B, S, D)))
        pos = pos2.reshape(P2)
        be = be2.reshape(NB)
        xi = xi2.reshape(NB)
        actb = act2.reshape(NB)

        xs = dispatch(x, pos)
        if _STOP == 2:
            return jnp.sum(xs) + jnp.zeros((B, S, D))
        y = expert(be, xi, actb, xs, gate_up_proj, bgu, wd2, bd)
        if _STOP == 3:
            return y[:T].reshape(B, S, D)
        out0, out1 = combine(y, pos)
        out = mix(wp2[:T], wp2[T:], out0, out1)
        return out.reshape(B, S, D)

    return run


def kernel(hidden_states, router_weight, router_bias, gate_up_proj,
           gate_up_proj_bias, down_proj, down_proj_bias):
    return _build(False)(hidden_states, router_weight, router_bias,
                         gate_up_proj, gate_up_proj_bias, down_proj,
                         down_proj_bias)


# speedup vs baseline: 1.8125x; 1.8125x over previous
import functools

import jax
import jax.numpy as jnp
from jax import lax
from jax.experimental import pallas as pl
from jax.experimental.pallas import tpu as pltpu
from jax.experimental.pallas import tpu_sc as plsc

B, S, D = 1, 2048, 768
E, K, F = 8, 2, 768
ALPHA, LIMIT = 1.702, 7.0

T = B * S
P2 = 2 * T
BLK = 256
NROWS = P2 + E * BLK
NB = NROWS // BLK
CHUNK = 512

_STOP = 99

NSUB = 32
PAIRS_PER = P2 // NSUB
TOK_PER = T // NSUB
HALF = TOK_PER // 2


def _router_body(x_ref, wr_ref, br_ref, pos_ref, wp_ref, be_ref, xi_ref,
                 act_ref):
    x = x_ref[...]
    logits = jnp.dot(x, wr_ref[...], preferred_element_type=jnp.float32)
    logits = logits + br_ref[...]
    ids = lax.broadcasted_iota(jnp.int32, (T, E), 1)
    m1 = jnp.max(logits, axis=1, keepdims=True)
    i1 = jnp.min(jnp.where(logits == m1, ids, E), axis=1, keepdims=True)
    mask1 = ids == i1
    l2 = jnp.where(mask1, -jnp.inf, logits)
    m2 = jnp.max(l2, axis=1, keepdims=True)
    i2 = jnp.min(jnp.where(l2 == m2, ids, E), axis=1, keepdims=True)
    mask2 = ids == i2
    tt = jnp.exp(m2 - m1)
    w1 = 1.0 / (1.0 + tt)
    w2 = 1.0 - w1

    oh = jnp.concatenate([mask1, mask2], axis=0).astype(jnp.float32)
    wp_ref[...] = jnp.concatenate([w1, w2], axis=0)

    ci = lax.broadcasted_iota(jnp.int32, (CHUNK, CHUNK), 0)
    cj = lax.broadcasted_iota(jnp.int32, (CHUNK, CHUNK), 1)
    tri = (cj <= ci).astype(jnp.float32)
    run = jnp.zeros((1, E), jnp.float32)
    chunks = []
    for c in range(P2 // CHUNK):
        seg = oh[c * CHUNK:(c + 1) * CHUNK]
        cs = jnp.dot(tri, seg, preferred_element_type=jnp.float32) + run
        run = cs[CHUNK - 1:CHUNK, :]
        chunks.append(cs)
    cums = jnp.concatenate(chunks, axis=0)

    counts = run.astype(jnp.int32)
    padded = ((counts + (BLK - 1)) // BLK) * BLK
    ei = lax.broadcasted_iota(jnp.int32, (E, E), 0)
    ej = lax.broadcasted_iota(jnp.int32, (E, E), 1)
    mex = (ei < ej).astype(jnp.float32)
    offs = jnp.dot(padded.astype(jnp.float32), mex,
                   preferred_element_type=jnp.float32)
    pos_ref[...] = jnp.sum(oh * (offs + cums - 1.0), axis=1,
                           keepdims=True).astype(jnp.int32)

    total = jnp.sum(padded)
    n_active = total // BLK
    bi = lax.broadcasted_iota(jnp.int32, (NB, 1), 0)
    bstart = bi * BLK
    offs_b = jnp.broadcast_to(offs.astype(jnp.int32), (NB, E))
    be = jnp.sum((offs_b <= bstart).astype(jnp.int32), axis=1,
                 keepdims=True) - 1
    act = (bstart < total).astype(jnp.int32)
    be_last = jnp.max(jnp.where(act == 1, be, -1), axis=0, keepdims=True)
    be_ref[...] = jnp.where(act == 1, be, be_last)
    xi_ref[...] = jnp.minimum(bi, n_active - 1)
    act_ref[...] = act


def _expert_body(be_ref, xi_ref, act_ref, xs_ref, wgu_ref, bgu_ref, wd2_ref,
                 bd_ref, y_ref):
    b = pl.program_id(0)

    @pl.when(act_ref[b] == 1)
    def _():
        x = xs_ref[...].astype(jnp.bfloat16)
        gu = jnp.dot(x, wgu_ref[0], preferred_element_type=jnp.float32)
        gu = gu + bgu_ref[0]
        g = pltpu.roll(gu, 1, 1)
        g = jnp.minimum(g, LIMIT)
        u = jnp.clip(gu, -LIMIT, LIMIT)
        actv = (u + 1.0) * (g * jax.nn.sigmoid(g * ALPHA))
        y = jnp.dot(actv.astype(jnp.bfloat16), wd2_ref[0],
                    preferred_element_type=jnp.float32)
        y_ref[...] = y + bd_ref[0]


def _cast_gu_body(w_ref, o_ref):
    o_ref[...] = w_ref[...].astype(jnp.bfloat16)


def _cast_dp_body(w_ref, o_ref):
    w = w_ref[0].astype(jnp.bfloat16)
    z = jnp.zeros((F, D), jnp.bfloat16)
    o_ref[0] = jnp.stack([z, w], axis=1).reshape(2 * F, D)


def _dispatch_body(x_hbm, pos_hbm, xs_hbm, src_v, pos_v, rows_v, sem):
    wid = lax.axis_index("s") * 2 + lax.axis_index("c")
    base = wid * PAIRS_PER
    for i in range(PAIRS_PER // 16):
        v = base + i * 16 + lax.broadcasted_iota(jnp.int32, (16,), 0)
        src_v[pl.ds(i * 16, 16)] = lax.rem(v, T)
    pltpu.sync_copy(pos_hbm.at[pl.ds(base, PAIRS_PER)], pos_v)
    pltpu.async_copy(x_hbm.at[src_v], rows_v, sem).wait()
    pltpu.async_copy(rows_v, xs_hbm.at[pos_v], sem).wait()


def _combine_body(y_hbm, pos_hbm, out0_hbm, out1_hbm, pos_v, rows_v, sem):
    wid = lax.axis_index("s") * 2 + lax.axis_index("c")
    t0 = wid * TOK_PER
    pltpu.sync_copy(pos_hbm.at[pl.ds(t0, TOK_PER)], pos_v)
    pltpu.async_copy(y_hbm.at[pos_v], rows_v, sem).wait()
    pltpu.sync_copy(rows_v, out0_hbm.at[pl.ds(t0, TOK_PER)])
    pltpu.sync_copy(pos_hbm.at[pl.ds(T + t0, TOK_PER)], pos_v)
    pltpu.async_copy(y_hbm.at[pos_v], rows_v, sem).wait()
    pltpu.sync_copy(rows_v, out1_hbm.at[pl.ds(t0, TOK_PER)])


def _mix_body(w0_ref, w1_ref, a_ref, b_ref, o_ref):
    o_ref[...] = w0_ref[...] * a_ref[...] + w1_ref[...] * b_ref[...]


@functools.lru_cache(maxsize=2)
def _build(interpret: bool = False):
    mesh = plsc.VectorSubcoreMesh(core_axis_name="c", subcore_axis_name="s")

    router = pl.pallas_call(
        _router_body,
        out_shape=[
            jax.ShapeDtypeStruct((P2, 1), jnp.int32),
            jax.ShapeDtypeStruct((P2, 1), jnp.float32),
            jax.ShapeDtypeStruct((NB, 1), jnp.int32),
            jax.ShapeDtypeStruct((NB, 1), jnp.int32),
            jax.ShapeDtypeStruct((NB, 1), jnp.int32),
        ],
        interpret=interpret,
    )

    dispatch = functools.partial(
        pl.kernel,
        out_type=jax.ShapeDtypeStruct((NROWS, D), jnp.float32),
        mesh=mesh,
        scratch_types=[
            pltpu.VMEM((PAIRS_PER,), jnp.int32),
            pltpu.VMEM((PAIRS_PER,), jnp.int32),
            pltpu.VMEM((PAIRS_PER, D), jnp.float32),
            pltpu.SemaphoreType.DMA,
        ],
        interpret=interpret,
    )(_dispatch_body)

    grid_spec = pltpu.PrefetchScalarGridSpec(
        num_scalar_prefetch=3,
        grid=(NB,),
        in_specs=[
            pl.BlockSpec((BLK, D), lambda b, be, xi, act: (xi[b], 0)),
            pl.BlockSpec((1, D, 2 * F), lambda b, be, xi, act: (be[b], 0, 0)),
            pl.BlockSpec((1, 1, 2 * F), lambda b, be, xi, act: (be[b], 0, 0)),
            pl.BlockSpec((1, 2 * F, D), lambda b, be, xi, act: (be[b], 0, 0)),
            pl.BlockSpec((1, 1, D), lambda b, be, xi, act: (be[b], 0, 0)),
        ],
        out_specs=pl.BlockSpec((BLK, D), lambda b, be, xi, act: (b, 0)),
    )
    expert = pl.pallas_call(
        _expert_body,
        grid_spec=grid_spec,
        out_shape=jax.ShapeDtypeStruct((NROWS, D), jnp.float32),
        interpret=interpret,
    )

    combine = functools.partial(
        pl.kernel,
        out_type=[
            jax.ShapeDtypeStruct((T, D), jnp.float32),
            jax.ShapeDtypeStruct((T, D), jnp.float32),
        ],
        mesh=mesh,
        scratch_types=[
            pltpu.VMEM((TOK_PER,), jnp.int32),
            pltpu.VMEM((TOK_PER, D), jnp.float32),
            pltpu.SemaphoreType.DMA,
        ],
        interpret=interpret,
    )(_combine_body)

    cast_gu = pl.pallas_call(
        _cast_gu_body,
        grid=(E,),
        in_specs=[pl.BlockSpec((1, D, 2 * F), lambda e: (e, 0, 0))],
        out_specs=pl.BlockSpec((1, D, 2 * F), lambda e: (e, 0, 0)),
        out_shape=jax.ShapeDtypeStruct((E, D, 2 * F), jnp.bfloat16),
        interpret=interpret,
    )

    cast_dp = pl.pallas_call(
        _cast_dp_body,
        grid=(E,),
        in_specs=[pl.BlockSpec((1, F, D), lambda e: (e, 0, 0))],
        out_specs=pl.BlockSpec((1, 2 * F, D), lambda e: (e, 0, 0)),
        out_shape=jax.ShapeDtypeStruct((E, 2 * F, D), jnp.bfloat16),
        interpret=interpret,
    )

    mix = pl.pallas_call(
        _mix_body,
        out_shape=jax.ShapeDtypeStruct((T, D), jnp.float32),
        interpret=interpret,
    )

    def run(hidden_states, router_weight, router_bias, gate_up_proj,
            gate_up_proj_bias, down_proj, down_proj_bias):
        x = hidden_states.reshape(T, D)
        bgu = gate_up_proj_bias.reshape(E, 1, 2 * F)
        wgu16 = cast_gu(gate_up_proj)
        wd2 = cast_dp(down_proj)
        bd = down_proj_bias.reshape(E, 1, D)
        br = router_bias.reshape(1, E)

        pos2, wp2, be2, xi2, act2 = router(x, router_weight, br)
        if _STOP == 1:
            return (jnp.sum(wd2) + jnp.sum(pos2.astype(jnp.float32)) +
                    jnp.zeros((B, S, D)))
        pos = pos2.reshape(P2)
        be = be2.reshape(NB)
        xi = xi2.reshape(NB)
        actb = act2.reshape(NB)

        xs = dispatch(x, pos)
        if _STOP == 2:
            return jnp.sum(xs) + jnp.zeros((B, S, D))
        y = expert(be, xi, actb, xs, wgu16, bgu, wd2, bd)
        if _STOP == 3:
            return y[:T].reshape(B, S, D)
        out0, out1 = combine(y, pos)
        out = mix(wp2[:T], wp2[T:], out0, out1)
        return out.reshape(B, S, D)

    return run


def kernel(hidden_states, router_weight, router_bias, gate_up_proj,
           gate_up_proj_bias, down_proj, down_proj_bias):
    return _build(False)(hidden_states, router_weight, router_bias,
                         gate_up_proj, gate_up_proj_bias, down_proj,
                         down_proj_bias)
